# Initial kernel scaffold; baseline (speedup 1.0000x reference)
import jax
import jax.numpy as jnp
from jax.experimental import pallas as pl


def kernel(x, edge_index, batch, global_attr, W1, b1, W2, b2, Wfc1, bfc1, Wfc2, bfc2):
    raise NotImplementedError("write your pallas kernel here")


# speedup vs baseline: 23.8752x; 23.8752x over previous
import functools

import jax
import jax.numpy as jnp
from jax import lax
from jax.experimental import pallas as pl
from jax.experimental.pallas import tpu as pltpu
from jax.experimental.pallas import tpu_sc as plsc

N = 10000
E = 320000
D = 128
H = 64
G = 64
GF = 16

NC = 2
NS = 16
NW = NC * NS
EPW = E // NW
K = 80
NCHUNK = EPW // K
NPAD = 10240
RPT = NPAD // NS

_mesh = plsc.VectorSubcoreMesh(
    core_axis_name="c", subcore_axis_name="s", num_cores=NC, num_subcores=NS)


def _deg_body(dst_hbm, zeros1_hbm, out_hbm, dstv, ones_v, acc, sem):
    c = lax.axis_index("c")
    s = lax.axis_index("s")
    w = c * NS + s
    pltpu.sync_copy(dst_hbm.at[w], dstv)
    for i in range(K // 16):
        ones_v[pl.ds(i * 16, 16)] = jnp.ones((16,), jnp.float32)
    r0 = s * RPT
    pltpu.sync_copy(zeros1_hbm.at[pl.ds(r0, RPT)], acc.at[pl.ds(r0, RPT)])
    plsc.subcore_barrier()

    def step(j, carry):
        pltpu.sync_copy(ones_v, acc.at[dstv.at[j]], add=True)
        return carry

    lax.fori_loop(0, NCHUNK, step, 0)
    plsc.subcore_barrier()
    pltpu.sync_copy(acc.at[pl.ds(r0, RPT)], out_hbm.at[c, pl.ds(r0, RPT)])


_deg_kernel = pl.kernel(
    _deg_body,
    out_type=jax.ShapeDtypeStruct((NC, NPAD), jnp.float32),
    mesh=_mesh,
    scratch_types=[
        pltpu.VMEM((NCHUNK, K), jnp.int32),
        pltpu.VMEM((K,), jnp.float32),
        pltpu.VMEM_SHARED((NPAD,), jnp.float32),
        pltpu.SemaphoreType.DMA,
    ],
)


def _agg_body(g_hbm, src_hbm, dst_hbm, zeros2_hbm, out_hbm,
              srcv, dstv, buf, acc, sem):
    c = lax.axis_index("c")
    s = lax.axis_index("s")
    w = c * NS + s
    pltpu.sync_copy(src_hbm.at[w], srcv)
    pltpu.sync_copy(dst_hbm.at[w], dstv)
    r0 = s * RPT
    pltpu.sync_copy(zeros2_hbm.at[pl.ds(r0, RPT)], acc.at[pl.ds(r0, RPT)])
    plsc.subcore_barrier()

    def step(j, carry):
        pltpu.async_copy(g_hbm.at[srcv.at[j]], buf, sem).wait()
        pltpu.sync_copy(buf, acc.at[dstv.at[j]], add=True)
        return carry

    lax.fori_loop(0, NCHUNK, step, 0)
    plsc.subcore_barrier()
    pltpu.sync_copy(acc.at[pl.ds(r0, RPT)], out_hbm.at[c, pl.ds(r0, RPT)])


_agg_kernel = pl.kernel(
    _agg_body,
    out_type=jax.ShapeDtypeStruct((NC, NPAD, H), jnp.float32),
    mesh=_mesh,
    compiler_params=pltpu.CompilerParams(use_tc_tiling_on_sc=False),
    scratch_types=[
        pltpu.VMEM((NCHUNK, K), jnp.int32),
        pltpu.VMEM((NCHUNK, K), jnp.int32),
        pltpu.VMEM((K, H), jnp.float32),
        pltpu.VMEM_SHARED((NPAD, H), jnp.float32),
        pltpu.SemaphoreType.DMA,
    ],
)


def _tc1_body(x_ref, w1_ref, da_ref, db_ref, g1_ref, dinv_ref):
    dinv = lax.rsqrt(da_ref[...] + db_ref[...] + 1.0)
    h1 = jnp.dot(x_ref[...], w1_ref[...], preferred_element_type=jnp.float32)
    g1_ref[...] = h1 * dinv
    dinv_ref[...] = dinv


def _tc2_body(g1_ref, aa_ref, ab_ref, dinv_ref, b1_ref, w2_ref, g2_ref):
    t = (g1_ref[...] + aa_ref[...] + ab_ref[...]) * dinv_ref[...] + b1_ref[...]
    t = jnp.maximum(t, 0.0)
    h2 = jnp.dot(t, w2_ref[...], preferred_element_type=jnp.float32)
    g2_ref[...] = h2 * dinv_ref[...]


def _tc3_body(g2_ref, aa_ref, ab_ref, dinv_ref, b2_ref, batch_ref, gat_ref,
              wp_ref, wg_ref, bf1_ref, wf2_ref, bf2_ref, out_ref):
    h = (g2_ref[...] + aa_ref[...] + ab_ref[...]) * dinv_ref[...] + b2_ref[...]
    h = jnp.maximum(h, 0.0)
    gid = lax.broadcasted_iota(jnp.int32, (G, N), 0)
    mask = (gid == batch_ref[...]).astype(jnp.float32)
    counts = jnp.sum(mask, axis=1, keepdims=True)
    pooled = jnp.dot(mask, h, preferred_element_type=jnp.float32)
    pooled = pooled / jnp.maximum(counts, 1.0)
    z = (jnp.dot(pooled, wp_ref[...], preferred_element_type=jnp.float32)
         + jnp.dot(gat_ref[...], wg_ref[...], preferred_element_type=jnp.float32)
         + bf1_ref[...])
    z = jnp.maximum(z, 0.0)
    out_ref[...] = (jnp.dot(z, wf2_ref[...], preferred_element_type=jnp.float32)
                    + bf2_ref[...])


def _tc_call(body, out_shape, *args):
    return pl.pallas_call(body, out_shape=out_shape)(*args)


def kernel(x, edge_index, batch, global_attr, W1, b1, W2, b2,
           Wfc1, bfc1, Wfc2, bfc2):
    src = edge_index[0].reshape(NW, NCHUNK, K)
    dst = edge_index[1].reshape(NW, NCHUNK, K)
    zeros1 = jnp.zeros((NPAD,), jnp.float32)
    zeros2 = jnp.zeros((NPAD, H), jnp.float32)

    degp = _deg_kernel(dst, zeros1)
    da = degp[0, :N].reshape(N, 1)
    db = degp[1, :N].reshape(N, 1)

    g1, dinv = _tc_call(
        _tc1_body,
        (jax.ShapeDtypeStruct((N, H), jnp.float32),
         jax.ShapeDtypeStruct((N, 1), jnp.float32)),
        x, W1, da, db)

    agg1 = _agg_kernel(g1, src, dst, zeros2)
    g2 = _tc_call(
        _tc2_body, jax.ShapeDtypeStruct((N, H), jnp.float32),
        g1, agg1[0, :N], agg1[1, :N], dinv, b1.reshape(1, H), W2)

    agg2 = _agg_kernel(g2, src, dst, zeros2)
    out = _tc_call(
        _tc3_body, jax.ShapeDtypeStruct((G, 1), jnp.float32),
        g2, agg2[0, :N], agg2[1, :N], dinv, b2.reshape(1, H),
        batch.reshape(1, N), global_attr,
        Wfc1[:H], Wfc1[H:], bfc1.reshape(1, 64), Wfc2, bfc2.reshape(1, 1))
    return out.reshape(G)


# speedup vs baseline: 40.7803x; 1.7081x over previous
import functools

import jax
import jax.numpy as jnp
from jax import lax
from jax.experimental import pallas as pl
from jax.experimental.pallas import tpu as pltpu
from jax.experimental.pallas import tpu_sc as plsc

N = 10000
E = 320000
D = 128
H = 64
G = 64
GF = 16

NC = 2
NS = 16
NW = NC * NS
EPW = E // NW
K = 80
NCHUNK = EPW // K
NPAD = 10240
RPT = NPAD // NS

_mesh = plsc.VectorSubcoreMesh(
    core_axis_name="c", subcore_axis_name="s", num_cores=NC, num_subcores=NS)


def _deg_body(dst_hbm, zeros1_hbm, out_hbm, dstv, ones_v, acc, sem):
    c = lax.axis_index("c")
    s = lax.axis_index("s")
    w = c * NS + s
    pltpu.sync_copy(dst_hbm.at[w], dstv)
    for i in range(K // 16):
        ones_v[pl.ds(i * 16, 16)] = jnp.ones((16,), jnp.float32)
    r0 = s * RPT
    pltpu.sync_copy(zeros1_hbm.at[pl.ds(r0, RPT)], acc.at[pl.ds(r0, RPT)])
    plsc.subcore_barrier()

    def step(j, carry):
        pltpu.sync_copy(ones_v, acc.at[dstv.at[j]], add=True)
        return carry

    lax.fori_loop(0, NCHUNK, step, 0)
    plsc.subcore_barrier()
    pltpu.sync_copy(acc.at[pl.ds(r0, RPT)], out_hbm.at[c, pl.ds(r0, RPT)])


_deg_kernel = pl.kernel(
    _deg_body,
    out_type=jax.ShapeDtypeStruct((NC, NPAD), jnp.float32),
    mesh=_mesh,
    scratch_types=[
        pltpu.VMEM((NCHUNK, K), jnp.int32),
        pltpu.VMEM((K,), jnp.float32),
        pltpu.VMEM_SHARED((NPAD,), jnp.float32),
        pltpu.SemaphoreType.DMA,
    ],
)


KA = 125
NCA = EPW // KA
RPA = N // NS


def _agg_body(g_hbm, src_hbm, dst_hbm, zeros2_hbm, out_hbm,
              srcv, dstv, b0, b1, b2, b3, acc,
              gs0, gs1, gs2, gs3, ss0, ss1, ss2, ss3):
    c = lax.axis_index("c")
    s = lax.axis_index("s")
    w = c * NS + s
    pltpu.sync_copy(src_hbm.at[w], srcv)
    pltpu.sync_copy(dst_hbm.at[w], dstv)
    r0 = s * RPA
    pltpu.sync_copy(zeros2_hbm.at[pl.ds(r0, RPA)], acc.at[pl.ds(r0, RPA)])
    plsc.subcore_barrier()

    def gf(j, buf, sem):
        pltpu.async_copy(g_hbm.at[srcv.at[j]], buf, sem)

    def gw(j, buf, sem):
        pltpu.make_async_copy(g_hbm.at[srcv.at[j]], buf, sem).wait()

    def sf(j, buf, sem):
        pltpu.async_copy(buf, acc.at[dstv.at[j]], sem, add=True)

    def sw(j, buf, sem):
        pltpu.make_async_copy(buf, acc.at[dstv.at[j]], sem).wait()

    gf(0, b0, gs0)
    gf(1, b1, gs1)

    def step(t, carry):
        j0 = 4 * t
        gw(j0, b0, gs0)
        gw(j0 + 1, b1, gs1)
        sf(j0, b0, ss0)
        sf(j0 + 1, b1, ss1)

        @pl.when(t > 0)
        def _():
            sw(j0 - 2, b2, ss2)
            sw(j0 - 1, b3, ss3)

        gf(j0 + 2, b2, gs2)
        gf(j0 + 3, b3, gs3)
        gw(j0 + 2, b2, gs2)
        gw(j0 + 3, b3, gs3)
        sf(j0 + 2, b2, ss2)
        sf(j0 + 3, b3, ss3)
        sw(j0, b0, ss0)
        sw(j0 + 1, b1, ss1)

        @pl.when(t < NCA // 4 - 1)
        def _():
            gf(j0 + 4, b0, gs0)
            gf(j0 + 5, b1, gs1)

        return carry

    lax.fori_loop(0, NCA // 4, step, 0)
    sw(NCA - 2, b2, ss2)
    sw(NCA - 1, b3, ss3)
    plsc.subcore_barrier()
    pltpu.sync_copy(acc.at[pl.ds(r0, RPA)], out_hbm.at[c, pl.ds(r0, RPA)])


_agg_kernel = pl.kernel(
    _agg_body,
    out_type=jax.ShapeDtypeStruct((NC, N, H), jnp.float32),
    mesh=_mesh,
    compiler_params=pltpu.CompilerParams(use_tc_tiling_on_sc=False),
    scratch_types=[
        pltpu.VMEM((NCA, KA), jnp.int32),
        pltpu.VMEM((NCA, KA), jnp.int32),
        pltpu.VMEM((KA, H), jnp.float32),
        pltpu.VMEM((KA, H), jnp.float32),
        pltpu.VMEM((KA, H), jnp.float32),
        pltpu.VMEM((KA, H), jnp.float32),
        pltpu.VMEM_SHARED((N, H), jnp.float32),
        pltpu.SemaphoreType.DMA,
        pltpu.SemaphoreType.DMA,
        pltpu.SemaphoreType.DMA,
        pltpu.SemaphoreType.DMA,
        pltpu.SemaphoreType.DMA,
        pltpu.SemaphoreType.DMA,
        pltpu.SemaphoreType.DMA,
        pltpu.SemaphoreType.DMA,
    ],
)


def _tc1_body(x_ref, w1_ref, da_ref, db_ref, g1_ref, dinv_ref):
    dinv = lax.rsqrt(da_ref[...] + db_ref[...] + 1.0)
    h1 = jnp.dot(x_ref[...], w1_ref[...], preferred_element_type=jnp.float32)
    g1_ref[...] = h1 * dinv
    dinv_ref[...] = dinv


def _tc2_body(g1_ref, agg_ref, dinv_ref, b1_ref, w2_ref, g2_ref):
    t = (g1_ref[...] + agg_ref[0] + agg_ref[1]) * dinv_ref[...] + b1_ref[...]
    t = jnp.maximum(t, 0.0)
    h2 = jnp.dot(t, w2_ref[...], preferred_element_type=jnp.float32)
    g2_ref[...] = h2 * dinv_ref[...]


def _tc3_body(g2_ref, agg_ref, dinv_ref, b2_ref, batch_ref, gat_ref,
              wp_ref, wg_ref, bf1_ref, wf2_ref, bf2_ref, out_ref):
    h = (g2_ref[...] + agg_ref[0] + agg_ref[1]) * dinv_ref[...] + b2_ref[...]
    h = jnp.maximum(h, 0.0)
    gid = lax.broadcasted_iota(jnp.int32, (G, N), 0)
    mask = (gid == batch_ref[...]).astype(jnp.float32)
    counts = jnp.sum(mask, axis=1, keepdims=True)
    pooled = jnp.dot(mask, h, preferred_element_type=jnp.float32)
    pooled = pooled / jnp.maximum(counts, 1.0)
    z = (jnp.dot(pooled, wp_ref[...], preferred_element_type=jnp.float32)
         + jnp.dot(gat_ref[...], wg_ref[...], preferred_element_type=jnp.float32)
         + bf1_ref[...])
    z = jnp.maximum(z, 0.0)
    out_ref[...] = (jnp.dot(z, wf2_ref[...], preferred_element_type=jnp.float32)
                    + bf2_ref[...])


def _tc_call(body, out_shape, *args):
    return pl.pallas_call(body, out_shape=out_shape)(*args)


def kernel(x, edge_index, batch, global_attr, W1, b1, W2, b2,
           Wfc1, bfc1, Wfc2, bfc2):
    srcd = edge_index[0].reshape(NW, NCHUNK, K)
    dstd = edge_index[1].reshape(NW, NCHUNK, K)
    srca = edge_index[0].reshape(NW, NCA, KA)
    dsta = edge_index[1].reshape(NW, NCA, KA)
    zeros1 = jnp.zeros((NPAD,), jnp.float32)
    zeros2 = jnp.zeros((N, H), jnp.float32)

    degp = _deg_kernel(dstd, zeros1)
    da = degp[0, :N].reshape(N, 1)
    db = degp[1, :N].reshape(N, 1)

    g1, dinv = _tc_call(
        _tc1_body,
        (jax.ShapeDtypeStruct((N, H), jnp.float32),
         jax.ShapeDtypeStruct((N, 1), jnp.float32)),
        x, W1, da, db)

    agg1 = _agg_kernel(g1, srca, dsta, zeros2)
    g2 = _tc_call(
        _tc2_body, jax.ShapeDtypeStruct((N, H), jnp.float32),
        g1, agg1, dinv, b1.reshape(1, H), W2)

    agg2 = _agg_kernel(g2, srca, dsta, zeros2)
    out = _tc_call(
        _tc3_body, jax.ShapeDtypeStruct((G, 1), jnp.float32),
        g2, agg2, dinv, b2.reshape(1, H),
        batch.reshape(1, N), global_attr,
        Wfc1[:H], Wfc1[H:], bfc1.reshape(1, 64), Wfc2, bfc2.reshape(1, 1))
    return out.reshape(G)


# speedup vs baseline: 48.5199x; 1.1898x over previous
import jax
import jax.numpy as jnp
from jax import lax
from jax.experimental import pallas as pl
from jax.experimental.pallas import tpu as pltpu
from jax.experimental.pallas import tpu_sc as plsc

N = 10000
E = 320000
D = 128
H = 64
G = 64
GF = 16

NC = 2
NS = 16
NW = NC * NS
EPW = E // NW
KA = 125
NCA = EPW // KA
RPA = N // NS
NPAD = 10240
RPT = NPAD // NS

_mesh = plsc.VectorSubcoreMesh(
    core_axis_name="c", subcore_axis_name="s", num_cores=NC, num_subcores=NS)


def _deg_body(dst_hbm, out_hbm, dstv, ones_v, zbuf, acc, sem):
    c = lax.axis_index("c")
    s = lax.axis_index("s")
    w = c * NS + s
    pltpu.sync_copy(dst_hbm.at[w], dstv)
    for i in range(8):
        ones_v[pl.ds(i * 16, 16)] = jnp.ones((16,), jnp.float32)
        zbuf[pl.ds(i * 16, 16)] = jnp.zeros((16,), jnp.float32)
    r0 = s * RPT
    for i in range(RPT // 128):
        pltpu.sync_copy(zbuf, acc.at[pl.ds(r0 + i * 128, 128)])
    plsc.subcore_barrier()

    ones_k = ones_v.at[pl.ds(0, KA)]

    def fire(j, carry):
        pltpu.async_copy(ones_k, acc.at[dstv.at[j]], sem, add=True)
        return carry

    lax.fori_loop(0, NCA, fire, 0)

    def drain(j, carry):
        pltpu.make_async_copy(ones_k, acc.at[dstv.at[j]], sem).wait()
        return carry

    lax.fori_loop(0, NCA, drain, 0)
    plsc.subcore_barrier()
    pltpu.sync_copy(acc.at[pl.ds(r0, RPT)], out_hbm.at[c, pl.ds(r0, RPT)])


_deg_kernel = pl.kernel(
    _deg_body,
    out_type=jax.ShapeDtypeStruct((NC, NPAD), jnp.float32),
    mesh=_mesh,
    scratch_types=[
        pltpu.VMEM((NCA, KA), jnp.int32),
        pltpu.VMEM((128,), jnp.float32),
        pltpu.VMEM((128,), jnp.float32),
        pltpu.VMEM_SHARED((NPAD,), jnp.float32),
        pltpu.SemaphoreType.DMA,
    ],
)


def _agg_body(g_hbm, src_hbm, dst_hbm, out_hbm,
              srcv, dstv, b0, b1, b2, b3, b4, b5, b6, b7,
              g0, g1, g2, g3, g4, g5, g6, g7,
              s0, s1, s2, s3, s4, s5, s6, s7, acc):
    bufs = [b0, b1, b2, b3, b4, b5, b6, b7]
    gs = [g0, g1, g2, g3, g4, g5, g6, g7]
    ss = [s0, s1, s2, s3, s4, s5, s6, s7]
    c = lax.axis_index("c")
    s = lax.axis_index("s")
    w = c * NS + s
    pltpu.sync_copy(src_hbm.at[w], srcv)
    pltpu.sync_copy(dst_hbm.at[w], dstv)

    def zrow(i, carry):
        for kk in range(H // 16):
            b0[i, pl.ds(16 * kk, 16)] = jnp.zeros((16,), jnp.float32)
        return carry

    lax.fori_loop(0, KA, zrow, 0)
    r0 = s * RPA
    for i in range(RPA // KA):
        pltpu.sync_copy(b0, acc.at[pl.ds(r0 + i * KA, KA)])
    plsc.subcore_barrier()

    def gf(j, buf, sem):
        pltpu.async_copy(g_hbm.at[srcv.at[j]], buf, sem)

    def gw(j, buf, sem):
        pltpu.make_async_copy(g_hbm.at[srcv.at[j]], buf, sem).wait()

    def sf(j, buf, sem):
        pltpu.async_copy(buf, acc.at[dstv.at[j]], sem, add=True)

    def sw(j, buf, sem):
        pltpu.make_async_copy(buf, acc.at[dstv.at[j]], sem).wait()

    gf(0, bufs[0], gs[0])
    gf(1, bufs[1], gs[1])
    gf(2, bufs[2], gs[2])
    gf(3, bufs[3], gs[3])
    gf(4, bufs[4], gs[4])
    gf(5, bufs[5], gs[5])

    def step(t, carry):
        for k in range(4):
            j = 8 * t + 2 * k
            a0, a1 = 2 * k, 2 * k + 1
            p0, p1 = (2 * k - 2) % 8, (2 * k - 1) % 8
            gw(j, bufs[a0], gs[a0])
            gw(j + 1, bufs[a1], gs[a1])
            sf(j, bufs[a0], ss[a0])
            sf(j + 1, bufs[a1], ss[a1])

            def waits(j=j, p0=p0, p1=p1):
                sw(j - 2, bufs[p0], ss[p0])
                sw(j - 1, bufs[p1], ss[p1])

            def fires(j=j, p0=p0, p1=p1):
                gf(j + 6, bufs[p0], gs[p0])
                gf(j + 7, bufs[p1], gs[p1])

            if k == 0:
                pl.when(t > 0)(waits)
                fires()
            else:
                waits()
                pl.when(t < NCA // 8 - 1)(fires)
        return carry

    lax.fori_loop(0, NCA // 8, step, 0)
    sw(NCA - 2, bufs[6], ss[6])
    sw(NCA - 1, bufs[7], ss[7])
    plsc.subcore_barrier()
    pltpu.sync_copy(acc.at[pl.ds(r0, RPA)], out_hbm.at[c, pl.ds(r0, RPA)])


_agg_kernel = pl.kernel(
    _agg_body,
    out_type=jax.ShapeDtypeStruct((NC, N, H), jnp.float32),
    mesh=_mesh,
    compiler_params=pltpu.CompilerParams(use_tc_tiling_on_sc=False),
    scratch_types=(
        [pltpu.VMEM((NCA, KA), jnp.int32)] * 2
        + [pltpu.VMEM((KA, H), jnp.float32)] * 8
        + [pltpu.SemaphoreType.DMA] * 16
        + [pltpu.VMEM_SHARED((N, H), jnp.float32)]
    ),
)


def _tc1_body(x_ref, w1_ref, da_ref, db_ref, g1_ref, dinv_ref):
    dinv = lax.rsqrt(da_ref[...] + db_ref[...] + 1.0)
    h1 = jnp.dot(x_ref[...], w1_ref[...], preferred_element_type=jnp.float32)
    g1_ref[...] = h1 * dinv
    dinv_ref[...] = dinv


def _tc2_body(g1_ref, agg_ref, dinv_ref, b1_ref, w2_ref, g2_ref):
    t = (g1_ref[...] + agg_ref[0] + agg_ref[1]) * dinv_ref[...] + b1_ref[...]
    t = jnp.maximum(t, 0.0)
    h2 = jnp.dot(t, w2_ref[...], preferred_element_type=jnp.float32)
    g2_ref[...] = h2 * dinv_ref[...]


def _tc3_body(g2_ref, agg_ref, dinv_ref, b2_ref, batch_ref, gat_ref,
              wp_ref, wg_ref, bf1_ref, wf2_ref, bf2_ref, out_ref):
    h = (g2_ref[...] + agg_ref[0] + agg_ref[1]) * dinv_ref[...] + b2_ref[...]
    h = jnp.maximum(h, 0.0)
    gid = lax.broadcasted_iota(jnp.int32, (G, N), 0)
    mask = (gid == batch_ref[...]).astype(jnp.float32)
    counts = jnp.sum(mask, axis=1, keepdims=True)
    pooled = jnp.dot(mask, h, preferred_element_type=jnp.float32)
    pooled = pooled / jnp.maximum(counts, 1.0)
    z = (jnp.dot(pooled, wp_ref[...], preferred_element_type=jnp.float32)
         + jnp.dot(gat_ref[...], wg_ref[...], preferred_element_type=jnp.float32)
         + bf1_ref[...])
    z = jnp.maximum(z, 0.0)
    out_ref[...] = (jnp.dot(z, wf2_ref[...], preferred_element_type=jnp.float32)
                    + bf2_ref[...])


def _tc_call(body, out_shape, *args):
    return pl.pallas_call(body, out_shape=out_shape)(*args)


def kernel(x, edge_index, batch, global_attr, W1, b1, W2, b2,
           Wfc1, bfc1, Wfc2, bfc2):
    srca = edge_index[0].reshape(NW, NCA, KA)
    dsta = edge_index[1].reshape(NW, NCA, KA)

    degp = _deg_kernel(dsta)
    da = degp[0, :N].reshape(N, 1)
    db = degp[1, :N].reshape(N, 1)

    g1, dinv = _tc_call(
        _tc1_body,
        (jax.ShapeDtypeStruct((N, H), jnp.float32),
         jax.ShapeDtypeStruct((N, 1), jnp.float32)),
        x, W1, da, db)

    agg1 = _agg_kernel(g1, srca, dsta)
    g2 = _tc_call(
        _tc2_body, jax.ShapeDtypeStruct((N, H), jnp.float32),
        g1, agg1, dinv, b1.reshape(1, H), W2)

    agg2 = _agg_kernel(g2, srca, dsta)
    out = _tc_call(
        _tc3_body, jax.ShapeDtypeStruct((G, 1), jnp.float32),
        g2, agg2, dinv, b2.reshape(1, H),
        batch.reshape(1, N), global_attr,
        Wfc1[:H], Wfc1[H:], bfc1.reshape(1, 64), Wfc2, bfc2.reshape(1, 1))
    return out.reshape(G)
